# baseline (speedup 1.0000x reference)
import functools
import math

import jax
import jax.numpy as jnp
from jax import lax
from jax.experimental import pallas as pl
from jax.experimental.pallas import tpu as pltpu
from jax.experimental.pallas import tpu_sc as plsc

MAX_TOKENS = 1024
DIM = 1024
SCALE = 1.0 / math.sqrt(DIM)

NUM_CORES = 2
NUM_SUBCORES = 16
NUM_WORKERS = NUM_CORES * NUM_SUBCORES
LANES = 16

ELEMS = MAX_TOKENS * DIM
PER_WORKER = ELEMS // NUM_WORKERS
VECS = PER_WORKER // LANES
UNROLL = 8

_mesh = plsc.VectorSubcoreMesh(core_axis_name="c", subcore_axis_name="s")


@functools.partial(
    pl.kernel,
    mesh=_mesh,
    out_type=jax.ShapeDtypeStruct((ELEMS,), jnp.float32),
    scratch_types=[pltpu.VMEM((PER_WORKER,), jnp.float32)],
)
def _scaled_copy(embed_hbm, out_hbm, buf):
    wid = lax.axis_index("s") * NUM_CORES + lax.axis_index("c")
    base = wid * PER_WORKER
    pltpu.sync_copy(embed_hbm.at[pl.ds(base, PER_WORKER)], buf)

    def body(i, carry):
        off = i * (LANES * UNROLL)
        for u in range(UNROLL):
            sl = pl.ds(off + u * LANES, LANES)
            buf[sl] = buf[sl] * SCALE
        return carry

    lax.fori_loop(0, VECS // UNROLL, body, 0)
    pltpu.sync_copy(buf, out_hbm.at[pl.ds(base, PER_WORKER)])


def kernel(x, embed):
    del x
    out = _scaled_copy(embed.reshape(ELEMS))
    return out.reshape(MAX_TOKENS, DIM)


# speedup vs baseline: 1.0005x; 1.0005x over previous
import functools
import math

import jax
import jax.numpy as jnp
from jax import lax
from jax.experimental import pallas as pl
from jax.experimental.pallas import tpu as pltpu
from jax.experimental.pallas import tpu_sc as plsc

MAX_TOKENS = 1024
DIM = 1024
SCALE = 1.0 / math.sqrt(DIM)

NUM_CORES = 2
NUM_SUBCORES = 16
NUM_WORKERS = NUM_CORES * NUM_SUBCORES
LANES = 16

ELEMS = MAX_TOKENS * DIM
PER_WORKER = ELEMS // NUM_WORKERS
VECS = PER_WORKER // LANES
UNROLL = 8

_mesh = plsc.VectorSubcoreMesh(core_axis_name="c", subcore_axis_name="s")


@functools.partial(
    pl.kernel,
    mesh=_mesh,
    out_type=jax.ShapeDtypeStruct((ELEMS,), jnp.float32),
    scratch_types=[
        pltpu.VMEM((PER_WORKER,), jnp.float32),
        pltpu.VMEM((PER_WORKER,), jnp.float32),
    ],
)
def _scaled_copy(embed_hbm, out_hbm, buf_in, buf_out):
    wid = lax.axis_index("s") * NUM_CORES + lax.axis_index("c")
    base = wid * PER_WORKER
    pltpu.sync_copy(embed_hbm.at[pl.ds(base, PER_WORKER)], buf_in)

    @plsc.parallel_loop(0, VECS, unroll=UNROLL)
    def _(i):
        sl = pl.ds(i * LANES, LANES)
        buf_out[sl] = buf_in[sl] * SCALE

    pltpu.sync_copy(buf_out, out_hbm.at[pl.ds(base, PER_WORKER)])


def kernel(x, embed):
    del x
    out = _scaled_copy(embed.reshape(ELEMS))
    return out.reshape(MAX_TOKENS, DIM)


# speedup vs baseline: 1.2915x; 1.2909x over previous
import functools
import math

import jax
import jax.numpy as jnp
from jax import lax
from jax.experimental import pallas as pl
from jax.experimental.pallas import tpu as pltpu
from jax.experimental.pallas import tpu_sc as plsc

MAX_TOKENS = 1024
DIM = 1024
SCALE = 1.0 / math.sqrt(DIM)

NUM_CORES = 2
NUM_SUBCORES = 16
NUM_WORKERS = NUM_CORES * NUM_SUBCORES
LANES = 16

ROWS_PER_WORKER = MAX_TOKENS // NUM_WORKERS
VECS_PER_ROW = DIM // LANES

_mesh = plsc.VectorSubcoreMesh(core_axis_name="c", subcore_axis_name="s")


@functools.partial(
    pl.kernel,
    mesh=_mesh,
    out_type=jax.ShapeDtypeStruct((MAX_TOKENS, DIM), jnp.float32),
    scratch_types=[
        pltpu.VMEM((ROWS_PER_WORKER, DIM), jnp.float32),
        pltpu.VMEM((ROWS_PER_WORKER, DIM), jnp.float32),
    ],
)
def _scaled_copy(embed_hbm, out_hbm, buf_in, buf_out):
    wid = lax.axis_index("s") * NUM_CORES + lax.axis_index("c")
    base = wid * ROWS_PER_WORKER
    pltpu.sync_copy(embed_hbm.at[pl.ds(base, ROWS_PER_WORKER)], buf_in)

    @plsc.parallel_loop(0, ROWS_PER_WORKER)
    def _(r):
        for c in range(VECS_PER_ROW):
            sl = pl.ds(c * LANES, LANES)
            buf_out[r, sl] = buf_in[r, sl] * SCALE

    pltpu.sync_copy(buf_out, out_hbm.at[pl.ds(base, ROWS_PER_WORKER)])


def kernel(x, embed):
    del x
    return _scaled_copy(embed)
